# baseline (speedup 1.0000x reference)
import functools

import jax
import jax.numpy as jnp
from jax import lax
from jax.experimental import pallas as pl
from jax.experimental.pallas import tpu as pltpu
from jax.experimental.pallas import tpu_sc as plsc

_IP = 5888


def _tc_tables(qos):
    T, U, I = qos.shape

    UB = 48
    NU = (U + UB - 1) // UB
    UP = NU * UB

    def body(q_ref, s_ref, c_ref, s_scr, c_scr):
        t = pl.program_id(0)
        ub = pl.program_id(1)
        x = q_ref[0]
        nz = jnp.where(x > 0, 1.0, 0.0)
        usl = pl.ds(pl.multiple_of(ub * UB, 8), UB)

        @pl.when(t == 0)
        def _():
            s_scr[usl, :] = x
            c_scr[usl, :] = nz

        @pl.when(t != 0)
        def _():
            s_scr[usl, :] = s_scr[usl, :] + x
            c_scr[usl, :] = c_scr[usl, :] + nz

        @pl.when(jnp.logical_and(t == T - 1, ub == NU - 1))
        def _():
            def row(r, _):
                off = pl.multiple_of(r * _IP, 128)
                s_ref[pl.ds(off, I)] = s_scr[r]
                c_ref[pl.ds(off, I)] = c_scr[r]
                return 0

            lax.fori_loop(0, U, row, 0)

    out_sd = jax.ShapeDtypeStruct((U * _IP,), jnp.float32)
    return pl.pallas_call(
        body,
        grid=(T, NU),
        compiler_params=pltpu.CompilerParams(
            vmem_limit_bytes=100 * 1024 * 1024),
        in_specs=[pl.BlockSpec((1, UB, I), lambda t, ub: (t, ub, 0))],
        out_specs=[pl.BlockSpec((U * _IP,), lambda t, ub: (0,)),
                   pl.BlockSpec((U * _IP,), lambda t, ub: (0,))],
        out_shape=[out_sd, out_sd],
        scratch_shapes=[pltpu.VMEM((UP, I), jnp.float32),
                        pltpu.VMEM((UP, I), jnp.float32)],
    )(qos)


def _make_sc_curr(B, T, U, I):
    info = plsc.get_sparse_core_info()
    NC, NS, L = info.num_cores, info.num_subcores, info.num_lanes
    NW = NC * NS
    assert B % (8 * NW) == 0
    BPW = B // NW
    NF = 16

    mesh = plsc.VectorSubcoreMesh(core_axis_name="c", subcore_axis_name="s")

    @functools.partial(
        pl.kernel,
        mesh=mesh,
        out_type=jax.ShapeDtypeStruct((B,), jnp.float32),
        compiler_params=pltpu.CompilerParams(needs_layout_passes=False),
        scratch_types=[
            pltpu.VMEM((BPW,), jnp.int32),
            pltpu.VMEM((BPW,), jnp.int32),
            pltpu.VMEM((BPW,), jnp.int32),
            pltpu.VMEM((NF, 8, 128), jnp.float32),
            pltpu.VMEM((BPW,), jnp.float32),
        ] + [pltpu.SemaphoreType.DMA] * NF,
    )
    def sc_curr(uid_h, iid_h, tid_h, qos_h, cur_out_h,
                uid_v, iid_v, tid_v, tiles_v, cur_v, *sems):
        wid = lax.axis_index("s") * NC + lax.axis_index("c")
        q0 = wid * BPW
        pltpu.sync_copy(uid_h.at[pl.ds(q0, BPW)], uid_v)
        pltpu.sync_copy(iid_h.at[pl.ds(q0, BPW)], iid_v)
        pltpu.sync_copy(tid_h.at[pl.ds(q0, BPW)], tid_v)

        def wave(w, _):
            w0 = w * NF
            wsl = pl.ds(w0, L)
            uvec = uid_v[wsl]
            ivec = iid_v[wsl]
            tvec = tid_v[wsl]
            cps = []
            for c in range(NF):
                u0 = pl.multiple_of(uvec[c] & -8, 8)
                i0 = pl.multiple_of(ivec[c] & -128, 128)
                cps.append(pltpu.async_copy(
                    qos_h.at[tvec[c], pl.ds(u0, 8), pl.ds(i0, 128)],
                    tiles_v.at[c], sems[c]))
            for cp in cps:
                cp.wait()
            cur_v[wsl] = plsc.load_gather(
                tiles_v, [lax.iota(jnp.int32, L), uvec & 7, ivec & 127])
            return 0

        lax.fori_loop(0, BPW // NF, wave, 0)
        pltpu.sync_copy(cur_v, cur_out_h.at[pl.ds(q0, BPW)])

    return sc_curr


def _make_sc_combine(B):
    info = plsc.get_sparse_core_info()
    NC, NS, L = info.num_cores, info.num_subcores, info.num_lanes
    NW = NC * NS
    BPW = B // NW

    mesh = plsc.VectorSubcoreMesh(core_axis_name="c", subcore_axis_name="s")

    @functools.partial(
        pl.kernel,
        mesh=mesh,
        out_type=jax.ShapeDtypeStruct((B,), jnp.float32),
        compiler_params=pltpu.CompilerParams(needs_layout_passes=False),
        scratch_types=[
            pltpu.VMEM((BPW,), jnp.int32),
            pltpu.VMEM((BPW,), jnp.int32),
            pltpu.VMEM((BPW,), jnp.int32),
            pltpu.VMEM((BPW,), jnp.float32),
            pltpu.VMEM((BPW,), jnp.float32),
            pltpu.VMEM((BPW,), jnp.float32),
            pltpu.VMEM((BPW,), jnp.float32),
            pltpu.SemaphoreType.DMA,
            pltpu.SemaphoreType.DMA,
        ],
    )
    def sc_combine(uid_h, iid_h, cur_h, sum_h, cnt_h, out_h,
                   uid_v, iid_v, pix_v, s_v, c_v, cur_v, out_v,
                   sem_s, sem_c):
        wid = lax.axis_index("s") * NC + lax.axis_index("c")
        q0 = wid * BPW
        pltpu.sync_copy(uid_h.at[pl.ds(q0, BPW)], uid_v)
        pltpu.sync_copy(iid_h.at[pl.ds(q0, BPW)], iid_v)
        pltpu.sync_copy(cur_h.at[pl.ds(q0, BPW)], cur_v)

        for g in range(BPW // L):
            sl = pl.ds(g * L, L)
            pix_v[sl] = uid_v[sl] * _IP + iid_v[sl]

        cp_s = pltpu.async_copy(sum_h.at[pix_v], s_v, sem_s)
        cp_c = pltpu.async_copy(cnt_h.at[pix_v], c_v, sem_c)
        cp_s.wait()
        cp_c.wait()

        for g in range(BPW // L):
            sl = pl.ds(g * L, L)
            s = s_v[sl]
            c = c_v[sl]
            cur = cur_v[sl]
            s_o = s - cur
            c_o = c - jnp.where(cur > 0, 1.0, 0.0)
            out_v[sl] = jnp.where(c_o > 0, s_o / c_o, 0.0)

        pltpu.sync_copy(out_v, out_h.at[pl.ds(q0, BPW)])

    return sc_combine


def kernel(user_id, item_id, time_id, qos_tensor):
    T, U, I = qos_tensor.shape
    B = user_id.shape[0]
    uid = user_id.astype(jnp.int32)
    iid = item_id.astype(jnp.int32)
    tid = time_id.astype(jnp.int32)
    cur = _make_sc_curr(B, T, U, I)(uid, iid, tid, qos_tensor)
    sum_tab, cnt_tab = _tc_tables(qos_tensor)
    return _make_sc_combine(B)(uid, iid, cur, sum_tab, cnt_tab)


# speedup vs baseline: 1.3695x; 1.3695x over previous
import functools

import jax
import jax.numpy as jnp
from jax import lax
from jax.experimental import pallas as pl
from jax.experimental.pallas import tpu as pltpu
from jax.experimental.pallas import tpu_sc as plsc

_IP = 5888


def _tc_tables(qos):
    T, U, I = qos.shape

    NB = 2
    UP8 = ((U + 7) // 8) * 8

    def body(q_hbm, s_out, c_out, bufs, s_scr, c_scr, sems):
        for b in range(NB):
            pltpu.make_async_copy(q_hbm.at[b], bufs.at[b], sems.at[b]).start()

        def step(t, _):
            b = lax.rem(t, NB)
            pltpu.make_async_copy(q_hbm.at[t], bufs.at[b], sems.at[b]).wait()
            x = bufs[b]
            nz = jnp.where(x > 0, 1.0, 0.0)
            asl = (pl.ds(0, U), pl.ds(0, I))

            @pl.when(t == 0)
            def _():
                s_scr[asl] = x
                c_scr[asl] = nz

            @pl.when(t != 0)
            def _():
                s_scr[asl] = s_scr[asl] + x
                c_scr[asl] = c_scr[asl] + nz

            @pl.when(t + NB < T)
            def _():
                pltpu.make_async_copy(
                    q_hbm.at[t + NB], bufs.at[b], sems.at[b]).start()

            return 0

        lax.fori_loop(0, T, step, 0)

        def row(r, _):
            off = pl.multiple_of(r * _IP, 128)
            s_out[pl.ds(off, _IP)] = s_scr[r]
            c_out[pl.ds(off, _IP)] = c_scr[r]
            return 0

        lax.fori_loop(0, U, row, 0)

    out_sd = jax.ShapeDtypeStruct((UP8 * _IP,), jnp.float32)
    return pl.pallas_call(
        body,
        compiler_params=pltpu.CompilerParams(
            vmem_limit_bytes=100 * 1024 * 1024),
        in_specs=[pl.BlockSpec(memory_space=pl.ANY)],
        out_shape=[out_sd, out_sd],
        scratch_shapes=[pltpu.VMEM((NB, U, I), jnp.float32),
                        pltpu.VMEM((UP8, _IP), jnp.float32),
                        pltpu.VMEM((UP8, _IP), jnp.float32),
                        pltpu.SemaphoreType.DMA((NB,))],
    )(qos)


def _make_sc_curr(B, T, U, I):
    info = plsc.get_sparse_core_info()
    NC, NS, L = info.num_cores, info.num_subcores, info.num_lanes
    NW = NC * NS
    assert B % (8 * NW) == 0
    BPW = B // NW
    NF = 16

    mesh = plsc.VectorSubcoreMesh(core_axis_name="c", subcore_axis_name="s")

    @functools.partial(
        pl.kernel,
        mesh=mesh,
        out_type=jax.ShapeDtypeStruct((B,), jnp.float32),
        compiler_params=pltpu.CompilerParams(needs_layout_passes=False),
        scratch_types=[
            pltpu.VMEM((BPW,), jnp.int32),
            pltpu.VMEM((BPW,), jnp.int32),
            pltpu.VMEM((BPW,), jnp.int32),
            pltpu.VMEM((NF, 8, 128), jnp.float32),
            pltpu.VMEM((BPW,), jnp.float32),
        ] + [pltpu.SemaphoreType.DMA] * NF,
    )
    def sc_curr(uid_h, iid_h, tid_h, qos_h, cur_out_h,
                uid_v, iid_v, tid_v, tiles_v, cur_v, *sems):
        wid = lax.axis_index("s") * NC + lax.axis_index("c")
        q0 = wid * BPW
        pltpu.sync_copy(uid_h.at[pl.ds(q0, BPW)], uid_v)
        pltpu.sync_copy(iid_h.at[pl.ds(q0, BPW)], iid_v)
        pltpu.sync_copy(tid_h.at[pl.ds(q0, BPW)], tid_v)

        def wave(w, _):
            w0 = w * NF
            wsl = pl.ds(w0, L)
            uvec = uid_v[wsl]
            ivec = iid_v[wsl]
            tvec = tid_v[wsl]
            cps = []
            for c in range(NF):
                u0 = pl.multiple_of(uvec[c] & -8, 8)
                i0 = pl.multiple_of(ivec[c] & -128, 128)
                cps.append(pltpu.async_copy(
                    qos_h.at[tvec[c], pl.ds(u0, 8), pl.ds(i0, 128)],
                    tiles_v.at[c], sems[c]))
            for cp in cps:
                cp.wait()
            cur_v[wsl] = plsc.load_gather(
                tiles_v, [lax.iota(jnp.int32, L), uvec & 7, ivec & 127])
            return 0

        lax.fori_loop(0, BPW // NF, wave, 0)
        pltpu.sync_copy(cur_v, cur_out_h.at[pl.ds(q0, BPW)])

    return sc_curr


def _make_sc_combine(B):
    info = plsc.get_sparse_core_info()
    NC, NS, L = info.num_cores, info.num_subcores, info.num_lanes
    NW = NC * NS
    BPW = B // NW

    mesh = plsc.VectorSubcoreMesh(core_axis_name="c", subcore_axis_name="s")

    @functools.partial(
        pl.kernel,
        mesh=mesh,
        out_type=jax.ShapeDtypeStruct((B,), jnp.float32),
        compiler_params=pltpu.CompilerParams(needs_layout_passes=False),
        scratch_types=[
            pltpu.VMEM((BPW,), jnp.int32),
            pltpu.VMEM((BPW,), jnp.int32),
            pltpu.VMEM((BPW,), jnp.int32),
            pltpu.VMEM((BPW,), jnp.float32),
            pltpu.VMEM((BPW,), jnp.float32),
            pltpu.VMEM((BPW,), jnp.float32),
            pltpu.VMEM((BPW,), jnp.float32),
            pltpu.SemaphoreType.DMA,
            pltpu.SemaphoreType.DMA,
        ],
    )
    def sc_combine(uid_h, iid_h, cur_h, sum_h, cnt_h, out_h,
                   uid_v, iid_v, pix_v, s_v, c_v, cur_v, out_v,
                   sem_s, sem_c):
        wid = lax.axis_index("s") * NC + lax.axis_index("c")
        q0 = wid * BPW
        pltpu.sync_copy(uid_h.at[pl.ds(q0, BPW)], uid_v)
        pltpu.sync_copy(iid_h.at[pl.ds(q0, BPW)], iid_v)
        pltpu.sync_copy(cur_h.at[pl.ds(q0, BPW)], cur_v)

        for g in range(BPW // L):
            sl = pl.ds(g * L, L)
            pix_v[sl] = uid_v[sl] * _IP + iid_v[sl]

        cp_s = pltpu.async_copy(sum_h.at[pix_v], s_v, sem_s)
        cp_c = pltpu.async_copy(cnt_h.at[pix_v], c_v, sem_c)
        cp_s.wait()
        cp_c.wait()

        for g in range(BPW // L):
            sl = pl.ds(g * L, L)
            s = s_v[sl]
            c = c_v[sl]
            cur = cur_v[sl]
            s_o = s - cur
            c_o = c - jnp.where(cur > 0, 1.0, 0.0)
            out_v[sl] = jnp.where(c_o > 0, s_o / c_o, 0.0)

        pltpu.sync_copy(out_v, out_h.at[pl.ds(q0, BPW)])

    return sc_combine


def kernel(user_id, item_id, time_id, qos_tensor):
    T, U, I = qos_tensor.shape
    B = user_id.shape[0]
    uid = user_id.astype(jnp.int32)
    iid = item_id.astype(jnp.int32)
    tid = time_id.astype(jnp.int32)
    cur = _make_sc_curr(B, T, U, I)(uid, iid, tid, qos_tensor)
    sum_tab, cnt_tab = _tc_tables(qos_tensor)
    return _make_sc_combine(B)(uid, iid, cur, sum_tab, cnt_tab)


# speedup vs baseline: 1.3985x; 1.0212x over previous
import functools

import jax
import jax.numpy as jnp
from jax import lax
from jax.experimental import pallas as pl
from jax.experimental.pallas import tpu as pltpu
from jax.experimental.pallas import tpu_sc as plsc

_IP = 5888


def _tc_tables(qos):
    T, U, I = qos.shape

    NB = 3
    UP8 = ((U + 7) // 8) * 8
    NR8 = UP8 // 8

    def body(q_hbm, s_out, c_out, bufs, s_scr, c_scr, st_s, st_c,
             sems, rsems):
        for b in range(NB):
            pltpu.make_async_copy(q_hbm.at[b], bufs.at[b], sems.at[b]).start()

        def step(t, _):
            b = lax.rem(t, NB)
            pltpu.make_async_copy(q_hbm.at[t], bufs.at[b], sems.at[b]).wait()
            x = bufs[b]
            nz = jnp.where(x > 0, 1.0, 0.0)
            asl = (pl.ds(0, U), pl.ds(0, I))

            @pl.when(t == 0)
            def _():
                s_scr[asl] = x
                c_scr[asl] = nz

            @pl.when(t != 0)
            def _():
                s_scr[asl] = s_scr[asl] + x
                c_scr[asl] = c_scr[asl] + nz

            @pl.when(t + NB < T)
            def _():
                pltpu.make_async_copy(
                    q_hbm.at[t + NB], bufs.at[b], sems.at[b]).start()

            return 0

        lax.fori_loop(0, T, step, 0)

        def stage_dma(b, r8):
            off = pl.multiple_of(r8 * 8 * _IP, 128)
            cs = pltpu.make_async_copy(
                st_s.at[b], s_out.at[pl.ds(off, 8 * _IP)], rsems.at[b])
            cc = pltpu.make_async_copy(
                st_c.at[b], c_out.at[pl.ds(off, 8 * _IP)],
                rsems.at[b + 2])
            return cs, cc

        def rowblk(r8, _):
            b = lax.rem(r8, 2)

            @pl.when(r8 >= 2)
            def _():
                cs, cc = stage_dma(b, r8 - 2)
                cs.wait()
                cc.wait()

            for k in range(8):
                ksl = pl.ds(pl.multiple_of(k * _IP, 128), _IP)
                st_s[b, ksl] = s_scr[r8 * 8 + k]
                st_c[b, ksl] = c_scr[r8 * 8 + k]
            cs, cc = stage_dma(b, r8)
            cs.start()
            cc.start()
            return 0

        lax.fori_loop(0, NR8, rowblk, 0)

        def drain(r8, _):
            cs, cc = stage_dma(lax.rem(r8, 2), r8)
            cs.wait()
            cc.wait()
            return 0

        lax.fori_loop(NR8 - 2, NR8, drain, 0)

    out_sd = jax.ShapeDtypeStruct((UP8 * _IP,), jnp.float32)
    return pl.pallas_call(
        body,
        compiler_params=pltpu.CompilerParams(
            vmem_limit_bytes=100 * 1024 * 1024),
        in_specs=[pl.BlockSpec(memory_space=pl.ANY)],
        out_specs=[pl.BlockSpec(memory_space=pl.ANY),
                   pl.BlockSpec(memory_space=pl.ANY)],
        out_shape=[out_sd, out_sd],
        scratch_shapes=[pltpu.VMEM((NB, U, I), jnp.float32),
                        pltpu.VMEM((UP8, _IP), jnp.float32),
                        pltpu.VMEM((UP8, _IP), jnp.float32),
                        pltpu.VMEM((2, 8 * _IP), jnp.float32),
                        pltpu.VMEM((2, 8 * _IP), jnp.float32),
                        pltpu.SemaphoreType.DMA((NB,)),
                        pltpu.SemaphoreType.DMA((4,))],
    )(qos)


def _make_sc_curr(B, T, U, I):
    info = plsc.get_sparse_core_info()
    NC, NS, L = info.num_cores, info.num_subcores, info.num_lanes
    NW = NC * NS
    assert B % (8 * NW) == 0
    BPW = B // NW
    NF = 16

    mesh = plsc.VectorSubcoreMesh(core_axis_name="c", subcore_axis_name="s")

    @functools.partial(
        pl.kernel,
        mesh=mesh,
        out_type=jax.ShapeDtypeStruct((B,), jnp.float32),
        compiler_params=pltpu.CompilerParams(needs_layout_passes=False),
        scratch_types=[
            pltpu.VMEM((BPW,), jnp.int32),
            pltpu.VMEM((BPW,), jnp.int32),
            pltpu.VMEM((BPW,), jnp.int32),
            pltpu.VMEM((NF, 8, 128), jnp.float32),
            pltpu.VMEM((BPW,), jnp.float32),
        ] + [pltpu.SemaphoreType.DMA] * NF,
    )
    def sc_curr(uid_h, iid_h, tid_h, qos_h, cur_out_h,
                uid_v, iid_v, tid_v, tiles_v, cur_v, *sems):
        wid = lax.axis_index("s") * NC + lax.axis_index("c")
        q0 = wid * BPW
        pltpu.sync_copy(uid_h.at[pl.ds(q0, BPW)], uid_v)
        pltpu.sync_copy(iid_h.at[pl.ds(q0, BPW)], iid_v)
        pltpu.sync_copy(tid_h.at[pl.ds(q0, BPW)], tid_v)

        def wave(w, _):
            w0 = w * NF
            wsl = pl.ds(w0, L)
            uvec = uid_v[wsl]
            ivec = iid_v[wsl]
            tvec = tid_v[wsl]
            cps = []
            for c in range(NF):
                u0 = pl.multiple_of(uvec[c] & -8, 8)
                i0 = pl.multiple_of(ivec[c] & -128, 128)
                cps.append(pltpu.async_copy(
                    qos_h.at[tvec[c], pl.ds(u0, 8), pl.ds(i0, 128)],
                    tiles_v.at[c], sems[c]))
            for cp in cps:
                cp.wait()
            cur_v[wsl] = plsc.load_gather(
                tiles_v, [lax.iota(jnp.int32, L), uvec & 7, ivec & 127])
            return 0

        lax.fori_loop(0, BPW // NF, wave, 0)
        pltpu.sync_copy(cur_v, cur_out_h.at[pl.ds(q0, BPW)])

    return sc_curr


def _make_sc_combine(B):
    info = plsc.get_sparse_core_info()
    NC, NS, L = info.num_cores, info.num_subcores, info.num_lanes
    NW = NC * NS
    BPW = B // NW

    mesh = plsc.VectorSubcoreMesh(core_axis_name="c", subcore_axis_name="s")

    @functools.partial(
        pl.kernel,
        mesh=mesh,
        out_type=jax.ShapeDtypeStruct((B,), jnp.float32),
        compiler_params=pltpu.CompilerParams(needs_layout_passes=False),
        scratch_types=[
            pltpu.VMEM((BPW,), jnp.int32),
            pltpu.VMEM((BPW,), jnp.int32),
            pltpu.VMEM((BPW,), jnp.int32),
            pltpu.VMEM((BPW,), jnp.float32),
            pltpu.VMEM((BPW,), jnp.float32),
            pltpu.VMEM((BPW,), jnp.float32),
            pltpu.VMEM((BPW,), jnp.float32),
            pltpu.SemaphoreType.DMA,
            pltpu.SemaphoreType.DMA,
        ],
    )
    def sc_combine(uid_h, iid_h, cur_h, sum_h, cnt_h, out_h,
                   uid_v, iid_v, pix_v, s_v, c_v, cur_v, out_v,
                   sem_s, sem_c):
        wid = lax.axis_index("s") * NC + lax.axis_index("c")
        q0 = wid * BPW
        pltpu.sync_copy(uid_h.at[pl.ds(q0, BPW)], uid_v)
        pltpu.sync_copy(iid_h.at[pl.ds(q0, BPW)], iid_v)
        pltpu.sync_copy(cur_h.at[pl.ds(q0, BPW)], cur_v)

        for g in range(BPW // L):
            sl = pl.ds(g * L, L)
            pix_v[sl] = uid_v[sl] * _IP + iid_v[sl]

        cp_s = pltpu.async_copy(sum_h.at[pix_v], s_v, sem_s)
        cp_c = pltpu.async_copy(cnt_h.at[pix_v], c_v, sem_c)
        cp_s.wait()
        cp_c.wait()

        for g in range(BPW // L):
            sl = pl.ds(g * L, L)
            s = s_v[sl]
            c = c_v[sl]
            cur = cur_v[sl]
            s_o = s - cur
            c_o = c - jnp.where(cur > 0, 1.0, 0.0)
            out_v[sl] = jnp.where(c_o > 0, s_o / c_o, 0.0)

        pltpu.sync_copy(out_v, out_h.at[pl.ds(q0, BPW)])

    return sc_combine


def kernel(user_id, item_id, time_id, qos_tensor):
    T, U, I = qos_tensor.shape
    B = user_id.shape[0]
    uid = user_id.astype(jnp.int32)
    iid = item_id.astype(jnp.int32)
    tid = time_id.astype(jnp.int32)
    cur = _make_sc_curr(B, T, U, I)(uid, iid, tid, qos_tensor)
    sum_tab, cnt_tab = _tc_tables(qos_tensor)
    return _make_sc_combine(B)(uid, iid, cur, sum_tab, cnt_tab)
